# baseline (speedup 1.0000x reference)
import functools

import jax
import jax.numpy as jnp
from jax import lax
from jax.experimental import pallas as pl
from jax.experimental.pallas import tpu as pltpu
from jax.experimental.pallas import tpu_sc as plsc

L = 16
NC = 2
NS = 16
SLOTS = 8


def _sc_mesh():
  return plsc.VectorSubcoreMesh(
      core_axis_name="c", subcore_axis_name="s", num_cores=NC, num_subcores=NS)


def _stage1_scores_copy(w_ref, x_ref, o_ref, s_ref):
  xb = x_ref[...]
  o_ref[...] = xb
  s_ref[...] = jnp.dot(xb, w_ref[...], preferred_element_type=jnp.float32)


def _stage4_layer(sel_ref, w_ref, b_ref, v_ref, o_ref):
  sb = sel_ref[...]
  p = jnp.dot(sb, w_ref[...], preferred_element_type=jnp.float32) + b_ref[...]
  sg = jax.nn.sigmoid(v_ref[...])
  o_ref[...] = sb + (p * sg - sb * sg)


def _make_topk(B, T, K, KP):
  CH = T // SLOTS
  NV = CH // L

  @functools.partial(
      pl.kernel,
      out_type=[
          jax.ShapeDtypeStruct((B * KP,), jnp.int32),
          jax.ShapeDtypeStruct((B * KP,), jnp.float32),
      ],
      mesh=_sc_mesh(),
      compiler_params=pltpu.CompilerParams(needs_layout_passes=False),
      scratch_types=[
          pltpu.VMEM((CH,), jnp.float32),
          pltpu.VMEM((CH,), jnp.uint32),
          pltpu.VMEM((CH,), jnp.int32),
          pltpu.VMEM((SLOTS, 128), jnp.int32),
          pltpu.SMEM((64,), jnp.int32),
          pltpu.SemaphoreType.DMA,
      ],
  )
  def _topk(scores_hbm, rid_hbm, val_hbm, svals, ukeys, rowid, dstidx, smem,
            sem):
    c = lax.axis_index("c")
    s = lax.axis_index("s")
    bl = s // SLOTS
    slot = s % SLOTS
    batch = c * 2 + bl
    base_flat = batch * T + slot * CH

    pltpu.sync_copy(scores_hbm.at[pl.ds(base_flat, CH)], svals)

    iota = lax.iota(jnp.int32, L)

    def _init(j, _):
      v = svals[pl.ds(j * L, L)]
      bts = plsc.bitcast(v, jnp.uint32)
      neg = bts >= jnp.uint32(0x80000000)
      ukeys[pl.ds(j * L, L)] = jnp.where(neg, ~bts, bts | jnp.uint32(0x80000000))
      rowid[pl.ds(j * L, L)] = base_flat + j * L + iota
      return 0

    lax.fori_loop(0, NV, _init, 0)

    def _zb(i, _):
      smem[i] = 0
      return 0

    lax.fori_loop(0, 64, _zb, 0)
    plsc.subcore_barrier()

    z16 = jnp.zeros((L,), jnp.int32)
    leader = bl * SLOTS
    kk = jnp.int32(K)

    def bb(i, carry):
      lo, hi = carry
      cont = lo < hi
      mid = lo + ((hi - lo) >> jnp.uint32(1))

      def cb(j, acc):
        kv = ukeys[pl.ds(j * L, L)]
        return acc + plsc.all_reduce_population_count(kv > mid)

      local = jnp.max(lax.fori_loop(0, NV, cb, z16, unroll=4))
      plsc.fetch_and_add(smem.at[i], local, subcore_id=leader)
      plsc.subcore_barrier()
      cg = plsc.fetch_and_add(smem.at[i], 0, subcore_id=leader)
      smaller = cg < kk
      nlo = jnp.where(smaller, lo, mid + jnp.uint32(1))
      nhi = jnp.where(smaller, mid, hi)
      return (jnp.where(cont, nlo, lo), jnp.where(cont, nhi, hi))

    _, thr = lax.fori_loop(0, 32, bb,
                           (jnp.uint32(0), jnp.uint32(0xFFFFFFFF)))

    def cb2(j, acc):
      ag, ae = acc
      kv = ukeys[pl.ds(j * L, L)]
      return (ag + plsc.all_reduce_population_count(kv > thr),
              ae + plsc.all_reduce_population_count(kv == thr))

    agv, aev = lax.fori_loop(0, NV, cb2, (z16, z16), unroll=4)
    plsc.fetch_and_add(smem.at[32 + slot], jnp.max(agv), subcore_id=leader)
    plsc.fetch_and_add(smem.at[40 + slot], jnp.max(aev), subcore_id=leader)
    plsc.subcore_barrier()

    def pb(r, acc):
      G, gp, ep = acc
      gr = plsc.fetch_and_add(smem.at[32 + r], 0, subcore_id=leader)
      er = plsc.fetch_and_add(smem.at[40 + r], 0, subcore_id=leader)
      before = r < slot
      return (G + gr,
              gp + jnp.where(before, gr, 0),
              ep + jnp.where(before, er, 0))

    Gg, gtpfx, eqpfx = lax.fori_loop(
        0, SLOTS, pb, (jnp.int32(0), jnp.int32(0), jnp.int32(0)))
    need = kk - Gg
    my_off = gtpfx + jnp.minimum(eqpfx, need)
    dst_base = batch * KP
    dump0 = dst_base + kk

    def comp(j, carry):
      pos, eqc = carry
      kv = ukeys[pl.ds(j * L, L)]
      gtm = kv > thr
      eqm = kv == thr
      eq_i = eqm.astype(jnp.int32)
      incl_e = plsc.cumsum(eq_i)
      excl_e = incl_e - eq_i
      take_eq = jnp.logical_and(eqm, (eqc + excl_e) < need)
      takem = jnp.logical_or(gtm, take_eq)
      t_i = takem.astype(jnp.int32)
      incl_t = plsc.cumsum(t_i)
      excl_t = incl_t - t_i
      dst = jnp.where(takem, dst_base + my_off + pos + excl_t,
                      dump0 + (j % SLOTS) * L + iota)
      dstidx[j // SLOTS, pl.ds((j % SLOTS) * L, L)] = dst
      return (pos + jnp.max(incl_t), eqc + jnp.max(incl_e))

    lax.fori_loop(0, NV, comp, (jnp.int32(0), jnp.int32(0)))

    def sc8(ci, _):
      idxrow = dstidx.at[ci]
      pltpu.async_copy(svals.at[pl.ds(ci * 128, 128)],
                       val_hbm.at[idxrow], sem).wait()
      pltpu.async_copy(rowid.at[pl.ds(ci * 128, 128)],
                       rid_hbm.at[idxrow], sem).wait()
      return 0

    lax.fori_loop(0, SLOTS, sc8, 0)

  return _topk


def _make_gather(B, D, K, KP):
  RPT = (B * K) // (NC * NS)

  @functools.partial(
      pl.kernel,
      out_type=jax.ShapeDtypeStruct((B * K, D), jnp.float32),
      mesh=_sc_mesh(),
      scratch_types=[
          pltpu.VMEM((RPT,), jnp.int32),
          pltpu.VMEM((RPT, D), jnp.float32),
          pltpu.SemaphoreType.DMA,
      ],
  )
  def _gather(rid_hbm, x_hbm, sel_hbm, idx_v, rows_v, sem):
    c = lax.axis_index("c")
    s = lax.axis_index("s")
    t = c * NS + s
    batch = t // SLOTS
    seg = t % SLOTS
    pltpu.sync_copy(rid_hbm.at[pl.ds(batch * KP + seg * RPT, RPT)], idx_v)
    pltpu.async_copy(x_hbm.at[idx_v], rows_v, sem).wait()
    pltpu.sync_copy(rows_v, sel_hbm.at[pl.ds(t * RPT, RPT)])

  return _gather


def _make_scatter(B, D, K, KP):
  RPT = (B * K) // (NC * NS)

  @functools.partial(
      pl.kernel,
      out_type=(),
      mesh=_sc_mesh(),
      scratch_types=[
          pltpu.VMEM((RPT,), jnp.int32),
          pltpu.VMEM((RPT, D), jnp.float32),
          pltpu.SemaphoreType.DMA,
      ],
  )
  def _scatter(new_hbm, rid_hbm, out_hbm, idx_v, rows_v, sem):
    c = lax.axis_index("c")
    s = lax.axis_index("s")
    t = c * NS + s
    batch = t // SLOTS
    seg = t % SLOTS
    pltpu.sync_copy(rid_hbm.at[pl.ds(batch * KP + seg * RPT, RPT)], idx_v)
    pltpu.sync_copy(new_hbm.at[pl.ds(t * RPT, RPT)], rows_v)
    pltpu.async_copy(rows_v, out_hbm.at[idx_v], sem).wait()

  return _scatter


def kernel(x, w_router, W_layer, b_layer):
  B, T, D = x.shape
  K = max(1, int(T * 0.125))
  KP = K + 128
  R = B * T

  RB = 1024
  x2d_in = x.reshape(R, D)
  out2, scores2 = pl.pallas_call(
      _stage1_scores_copy,
      out_shape=[
          jax.ShapeDtypeStruct((R, D), jnp.float32),
          jax.ShapeDtypeStruct((R, 1), jnp.float32),
      ],
      grid=(R // RB,),
      in_specs=[
          pl.BlockSpec((D, 1), lambda i: (0, 0)),
          pl.BlockSpec((RB, D), lambda i: (i, 0)),
      ],
      out_specs=[
          pl.BlockSpec((RB, D), lambda i: (i, 0)),
          pl.BlockSpec((RB, 1), lambda i: (i, 0)),
      ],
  )(w_router.reshape(D, 1), x2d_in)
  scores_flat = scores2.reshape(R)
  out3 = out2

  rid_pad, val_pad = _make_topk(B, T, K, KP)(scores_flat)

  x2d = x.reshape(R, D)
  sel = _make_gather(B, D, K, KP)(rid_pad, x2d)

  vals2 = val_pad.reshape(B, KP)[:, :K].reshape(B * K, 1)
  RBM = 512
  newrows = pl.pallas_call(
      _stage4_layer,
      out_shape=jax.ShapeDtypeStruct((B * K, D), jnp.float32),
      grid=((B * K) // RBM,),
      in_specs=[
          pl.BlockSpec((RBM, D), lambda i: (i, 0)),
          pl.BlockSpec((D, D), lambda i: (0, 0)),
          pl.BlockSpec((1, D), lambda i: (0, 0)),
          pl.BlockSpec((RBM, 1), lambda i: (i, 0)),
      ],
      out_specs=pl.BlockSpec((RBM, D), lambda i: (i, 0)),
  )(sel, W_layer, b_layer.reshape(1, D), vals2)

  out_ref = jax.new_ref(out3.reshape(R, D))
  _make_scatter(B, D, K, KP)(newrows, rid_pad, out_ref)

  return jax.freeze(out_ref).reshape(B, T, D)
